# baseline (speedup 1.0000x reference)
import functools

import jax
import jax.numpy as jnp
from jax import lax
from jax.experimental import pallas as pl
from jax.experimental.pallas import tpu as pltpu
from jax.experimental.pallas import tpu_sc as plsc

TILE = 512


def _gate_rank_body(x_ref, Wg_ref, bg_ref, tri_ref,
                    e1_ref, e2_ref, w1_ref, w2_ref, r1_ref, r2_ref,
                    off_ref, te_ref, carry_ref, oh2_ref, *, n_exp, tile, nb):
    i = pl.program_id(0)

    @pl.when(i == 0)
    def _init():
        carry_ref[...] = jnp.zeros_like(carry_ref)

    blk = x_ref.shape[0]

    @pl.when(i < nb)
    def _pass1():
        x = x_ref[...]
        logits = jnp.dot(x, Wg_ref[...], preferred_element_type=jnp.float32)
        logits = logits + bg_ref[...]
        iota_e = lax.broadcasted_iota(jnp.int32, logits.shape, 1)
        a1 = jnp.argmax(logits, axis=-1)[:, None]
        m1 = jnp.max(logits, axis=-1, keepdims=True)
        l2 = jnp.where(iota_e == a1, -jnp.inf, logits)
        a2 = jnp.argmax(l2, axis=-1)[:, None]
        m2 = jnp.max(l2, axis=-1, keepdims=True)
        w1 = 1.0 / (1.0 + jnp.exp(m2 - m1))
        e1_ref[...] = a1
        e2_ref[...] = a2
        w1_ref[...] = w1
        w2_ref[...] = 1.0 - w1
        oh1 = (iota_e == a1).astype(jnp.float32)
        oh2_ref[pl.ds(i * blk, blk), :] = (iota_e == a2).astype(jnp.float32)
        cum = jnp.dot(tri_ref[...], oh1, preferred_element_type=jnp.float32)
        cum = cum + carry_ref[...]
        r1_ref[...] = (jnp.sum(cum * oh1, axis=1, keepdims=True) - 1.0
                       ).astype(jnp.int32)
        carry_ref[...] = cum[-1:, :]

    @pl.when(i >= nb)
    def _pass2():
        oh2 = oh2_ref[pl.ds((i - nb) * blk, blk), :]
        cum = jnp.dot(tri_ref[...], oh2, preferred_element_type=jnp.float32)
        cum = cum + carry_ref[...]
        r2_ref[...] = (jnp.sum(cum * oh2, axis=1, keepdims=True) - 1.0
                       ).astype(jnp.int32)
        carry_new = cum[-1:, :]
        carry_ref[...] = carry_new
        pc = jnp.floor((carry_new + (tile - 1)) / tile) * tile
        r8 = lax.broadcasted_iota(jnp.int32, (n_exp, n_exp), 0)
        c8 = lax.broadcasted_iota(jnp.int32, (n_exp, n_exp), 1)
        incl_m = (r8 <= c8).astype(jnp.float32)
        incl = jnp.dot(pc, incl_m, preferred_element_type=jnp.float32)
        off_ref[...] = jnp.broadcast_to(incl - pc, off_ref.shape).astype(jnp.int32)
        nt_pad = te_ref.shape[0]
        ts = lax.broadcasted_iota(jnp.int32, (nt_pad, n_exp), 0) * tile
        acc = jnp.sum((ts.astype(jnp.float32) >= incl).astype(jnp.float32),
                      axis=1, keepdims=True).astype(jnp.int32)
        te_ref[...] = jnp.minimum(acc, n_exp - 1)


def _mk_route_build(P, T, n_pad, n_tiles, n_exp):
    mesh = plsc.VectorSubcoreMesh(core_axis_name="c", subcore_axis_name="s")

    @functools.partial(
        pl.kernel,
        mesh=mesh,
        out_type=[
            jax.ShapeDtypeStruct((n_pad,), jnp.int32),
            jax.ShapeDtypeStruct((n_pad,), jnp.float32),
            jax.ShapeDtypeStruct((P,), jnp.int32),
        ],
        compiler_params=pltpu.CompilerParams(needs_layout_passes=False),
        scratch_types=[
            pltpu.VMEM((P,), jnp.int32),
            pltpu.VMEM((P,), jnp.int32),
            pltpu.VMEM((P,), jnp.float32),
            pltpu.VMEM((P,), jnp.int32),
            pltpu.VMEM((n_pad,), jnp.int32),
            pltpu.VMEM((n_pad,), jnp.float32),
            pltpu.VMEM((16,), jnp.int32),
        ],
    )
    def route_build(e_hbm, r_hbm, w_hbm, off_hbm,
                    si_hbm, ws_hbm, pos_hbm,
                    e_v, r_v, w_v, pos_v, si_v, ws_v, off_v):
        wid = lax.axis_index("s") * 2 + lax.axis_index("c")

        @pl.when(wid == 0)
        def _work():
            pltpu.sync_copy(e_hbm, e_v)
            pltpu.sync_copy(r_hbm, r_v)
            pltpu.sync_copy(w_hbm, w_v)
            pltpu.sync_copy(off_hbm, off_v)

            lanes = lax.iota(jnp.int32, 16)

            def _zero(i, _):
                si_v[pl.ds(i * 16, 16)] = (i * 16 + lanes) & (T - 1)
                ws_v[pl.ds(i * 16, 16)] = jnp.zeros((16,), jnp.float32)
                return 0
            lax.fori_loop(0, n_pad // 16, _zero, 0)

            def _scatter(i, _):
                base = i * 16
                ve = e_v[pl.ds(base, 16)]
                vr = r_v[pl.ds(base, 16)]
                vw = w_v[pl.ds(base, 16)]
                voff = plsc.load_gather(off_v, [ve])
                vpos = voff + vr
                pos_v[pl.ds(base, 16)] = vpos
                vp = base + lanes
                vt = vp - jnp.where(vp >= T, jnp.int32(T), 0)
                plsc.store_scatter(si_v, [vpos], vt)
                plsc.store_scatter(ws_v, [vpos], vw)
                return 0
            lax.fori_loop(0, P // 16, _scatter, 0)

            pltpu.sync_copy(si_v, si_hbm)
            pltpu.sync_copy(ws_v, ws_hbm)
            pltpu.sync_copy(pos_v, pos_hbm)

    return route_build


def _mk_gather(T, D, n_pad, dtype=jnp.float32):
    mesh = plsc.VectorSubcoreMesh(core_axis_name="c", subcore_axis_name="s")
    NW = 32
    b_per_w = n_pad // NW
    CH = 64
    n_ch = b_per_w // CH

    @functools.partial(
        pl.kernel,
        mesh=mesh,
        out_type=jax.ShapeDtypeStruct((n_pad, D), dtype),
        compiler_params=pltpu.CompilerParams(needs_layout_passes=False),
        scratch_types=[
            pltpu.VMEM((b_per_w,), jnp.int32),
            pltpu.VMEM((CH, D), dtype),
            pltpu.VMEM((CH, D), dtype),
            pltpu.SemaphoreType.DMA,
            pltpu.SemaphoreType.DMA,
        ],
    )
    def gather_rows(x_hbm, idx_hbm, out_hbm, idx_v, b0_v, b1_v, s0, s1):
        wid = lax.axis_index("s") * 2 + lax.axis_index("c")
        base = wid * b_per_w
        pltpu.sync_copy(idx_hbm.at[pl.ds(base, b_per_w)], idx_v)

        def _chunk(c, _):
            cp0 = pltpu.async_copy(
                x_hbm.at[idx_v.at[pl.ds(2 * c * CH, CH)]], b0_v, s0)
            cp1 = pltpu.async_copy(
                x_hbm.at[idx_v.at[pl.ds((2 * c + 1) * CH, CH)]], b1_v, s1)
            cp0.wait()
            cp1.wait()
            pltpu.sync_copy(b0_v, out_hbm.at[pl.ds(base + 2 * c * CH, CH)])
            pltpu.sync_copy(b1_v, out_hbm.at[pl.ds(base + (2 * c + 1) * CH, CH)])
            return 0
        lax.fori_loop(0, n_ch // 2, _chunk, 0)

    return gather_rows


def _mk_route_gather(P, T, D, n_pad, n_exp):
    mesh = plsc.VectorSubcoreMesh(core_axis_name="c", subcore_axis_name="s")
    NW = 32
    b_per_w = n_pad // NW
    CH = 40
    CHK = 2048

    @functools.partial(
        pl.kernel,
        mesh=mesh,
        out_type=[
            jax.ShapeDtypeStruct((n_pad, D), jnp.float32),
            jax.ShapeDtypeStruct((n_pad,), jnp.float32),
            jax.ShapeDtypeStruct((P,), jnp.int32),
        ],
        compiler_params=pltpu.CompilerParams(needs_layout_passes=False),
        scratch_types=[
            pltpu.VMEM((16,), jnp.int32),
            pltpu.VMEM((CHK,), jnp.int32),
            pltpu.VMEM((CHK,), jnp.int32),
            pltpu.VMEM((CHK,), jnp.float32),
            pltpu.VMEM((CHK,), jnp.int32),
            pltpu.VMEM((n_pad,), jnp.int32),
            pltpu.VMEM((n_pad,), jnp.float32),
            pltpu.VMEM_SHARED((n_pad,), jnp.int32),
            pltpu.VMEM((b_per_w,), jnp.int32),
            pltpu.VMEM((CH, D), jnp.float32),
            pltpu.VMEM((CH, D), jnp.float32),
            pltpu.SemaphoreType.DMA,
            pltpu.SemaphoreType.DMA,
        ],
    )
    def route_gather(e_hbm, r_hbm, w_hbm, off_hbm, x_hbm,
                     xs_hbm, ws_hbm, pos_hbm,
                     off_v, e_c, r_c, w_c, pos_c, si_v, ws_v, sm_si,
                     idx_v, b0_v, b1_v, s0, s1):
        cid = lax.axis_index("c")
        sid = lax.axis_index("s")
        lanes = lax.iota(jnp.int32, 16)

        @pl.when(sid == 0)
        def _route():
            pltpu.sync_copy(off_hbm, off_v)

            def _pad_init(i, _):
                si_v[pl.ds(i * 16, 16)] = (i * 16 + lanes) & (T - 1)
                return 0
            lax.fori_loop(0, n_pad // 16, _pad_init, 0)

            def _chunk(k, _):
                pltpu.sync_copy(e_hbm.at[pl.ds(k * CHK, CHK)], e_c)
                pltpu.sync_copy(r_hbm.at[pl.ds(k * CHK, CHK)], r_c)
                pltpu.sync_copy(w_hbm.at[pl.ds(k * CHK, CHK)], w_c)

                def _scat(i, _):
                    base = i * 16
                    ve = e_c[pl.ds(base, 16)]
                    vr = r_c[pl.ds(base, 16)]
                    vw = w_c[pl.ds(base, 16)]
                    voff = plsc.load_gather(off_v, [ve])
                    vpos = voff + vr
                    pos_c[pl.ds(base, 16)] = vpos
                    vp = k * CHK + base + lanes
                    vt = vp - jnp.where(vp >= T, jnp.int32(T), 0)
                    plsc.store_scatter(si_v, [vpos], vt)
                    plsc.store_scatter(ws_v, [vpos], vw)
                    return 0
                lax.fori_loop(0, CHK // 16, _scat, 0)

                @pl.when(cid == 0)
                def _wpos():
                    pltpu.sync_copy(pos_c, pos_hbm.at[pl.ds(k * CHK, CHK)])
                return 0
            lax.fori_loop(0, P // CHK, _chunk, 0)
            pltpu.sync_copy(si_v, sm_si)

        plsc.subcore_barrier()

        @pl.when((sid == 0) & (cid == 0))
        def _wws():
            pltpu.sync_copy(ws_v, ws_hbm)

        wid = sid * 2 + cid
        base = wid * b_per_w
        pltpu.sync_copy(sm_si.at[pl.ds(base, b_per_w)], idx_v)

        def _g(c, _):
            cp0 = pltpu.async_copy(
                x_hbm.at[idx_v.at[pl.ds(2 * c * CH, CH)]], b0_v, s0)
            cp1 = pltpu.async_copy(
                x_hbm.at[idx_v.at[pl.ds((2 * c + 1) * CH, CH)]], b1_v, s1)
            cp0.wait()
            cp1.wait()
            pltpu.sync_copy(b0_v, xs_hbm.at[pl.ds(base + 2 * c * CH, CH)])
            pltpu.sync_copy(b1_v, xs_hbm.at[pl.ds(base + (2 * c + 1) * CH, CH)])
            return 0
        lax.fori_loop(0, b_per_w // (2 * CH), _g, 0)

    return route_gather


def _ffn_body(te_ref, xs_ref, W1_ref, b1_ref, W2_ref, b2_ref, ws_ref, ys_ref):
    x = xs_ref[...]
    h = jnp.dot(x, W1_ref[0], preferred_element_type=jnp.float32) + b1_ref[0]
    h = h * (1.0 / (1.0 + jnp.exp(-h)))
    y = jnp.dot(h, W2_ref[0], preferred_element_type=jnp.float32) + b2_ref[0]
    ys_ref[...] = y * ws_ref[...]


def _mk_combine(T, D, n_pad):
    mesh = plsc.VectorSubcoreMesh(core_axis_name="c", subcore_axis_name="s")
    NW = 32
    t_per_w = T // NW
    CH = 64
    n_ch = t_per_w // CH

    @functools.partial(
        pl.kernel,
        mesh=mesh,
        out_type=jax.ShapeDtypeStruct((T, D), jnp.float32),
        compiler_params=pltpu.CompilerParams(needs_layout_passes=False),
        scratch_types=[
            pltpu.VMEM((t_per_w,), jnp.int32),
            pltpu.VMEM((t_per_w,), jnp.int32),
            pltpu.VMEM((CH, D), jnp.float32),
            pltpu.VMEM((CH, D), jnp.float32),
            pltpu.SemaphoreType.DMA,
            pltpu.SemaphoreType.DMA,
        ],
    )
    def combine(ys_hbm, pos_hbm, out_hbm, i0_v, i1_v, b0_v, b1_v, s0, s1):
        wid = lax.axis_index("s") * 2 + lax.axis_index("c")
        base = wid * t_per_w
        pltpu.sync_copy(pos_hbm.at[pl.ds(base, t_per_w)], i0_v)
        pltpu.sync_copy(pos_hbm.at[pl.ds(T + base, t_per_w)], i1_v)

        def _chunk(c, _):
            cp0 = pltpu.async_copy(ys_hbm.at[i0_v.at[pl.ds(c * CH, CH)]], b0_v, s0)
            cp1 = pltpu.async_copy(ys_hbm.at[i1_v.at[pl.ds(c * CH, CH)]], b1_v, s1)
            cp0.wait()
            cp1.wait()

            def _row(r, _):
                for col in range(D // 16):
                    sl = pl.ds(col * 16, 16)
                    b0_v[r, sl] = b0_v[r, sl] + b1_v[r, sl]
                return 0
            lax.fori_loop(0, CH, _row, 0)
            pltpu.sync_copy(b0_v, out_hbm.at[pl.ds(base + c * CH, CH)])
            return 0
        lax.fori_loop(0, n_ch, _chunk, 0)

    return combine


def kernel(inputs, Wg, bg, W1, b1, W2, b2):
    B, S, D = inputs.shape
    E = Wg.shape[1]
    D_FF = W1.shape[2]
    T = B * S
    K = 2
    P = K * T
    N_TILES_MAX = (P + E * TILE) // TILE
    N_PAD = N_TILES_MAX * TILE
    x = inputs.reshape(T, D)

    BLK = 512
    NB = T // BLK
    NT_PAD = ((N_TILES_MAX + 15) // 16) * 16
    tri = jnp.tril(jnp.ones((BLK, BLK), jnp.float32))
    blk_map = lambda i: (jnp.minimum(i, NB - 1), 0)
    blk2_map = lambda i: (jnp.maximum(i - NB, 0), 0)
    fix_map = lambda i: (0, 0)
    e1, e2, w1, w2, rank1, rank2, off, tile_e = pl.pallas_call(
        functools.partial(_gate_rank_body, n_exp=E, tile=TILE, nb=NB),
        grid=(2 * NB,),
        in_specs=[
            pl.BlockSpec((BLK, D), blk_map),
            pl.BlockSpec((D, E), fix_map),
            pl.BlockSpec((1, E), fix_map),
            pl.BlockSpec((BLK, BLK), fix_map),
        ],
        out_specs=[
            pl.BlockSpec((BLK, 1), blk_map),
            pl.BlockSpec((BLK, 1), blk_map),
            pl.BlockSpec((BLK, 1), blk_map),
            pl.BlockSpec((BLK, 1), blk_map),
            pl.BlockSpec((BLK, 1), blk_map),
            pl.BlockSpec((BLK, 1), blk2_map),
            pl.BlockSpec((8, E), fix_map),
            pl.BlockSpec((NT_PAD, 1), fix_map),
        ],
        out_shape=[
            jax.ShapeDtypeStruct((T, 1), jnp.int32),
            jax.ShapeDtypeStruct((T, 1), jnp.int32),
            jax.ShapeDtypeStruct((T, 1), jnp.float32),
            jax.ShapeDtypeStruct((T, 1), jnp.float32),
            jax.ShapeDtypeStruct((T, 1), jnp.int32),
            jax.ShapeDtypeStruct((T, 1), jnp.int32),
            jax.ShapeDtypeStruct((8, E), jnp.int32),
            jax.ShapeDtypeStruct((NT_PAD, 1), jnp.int32),
        ],
        scratch_shapes=[
            pltpu.VMEM((1, E), jnp.float32),
            pltpu.VMEM((T, E), jnp.float32),
        ],
        compiler_params=pltpu.CompilerParams(
            dimension_semantics=("arbitrary",),
        ),
    )(x, Wg, bg.reshape(1, E), tri)

    e_all = jnp.concatenate([e1, e2], axis=0)
    w_all = jnp.concatenate([w1, w2], axis=0)
    rank = jnp.concatenate([rank1, rank2], axis=0)

    off16 = jnp.pad(off[0], (0, 16 - E))

    xs, wsort, pos = _mk_route_gather(P, T, D, N_PAD, E)(
        e_all.reshape(P), rank.reshape(P), w_all.reshape(P), off16, x)
    tile_e = tile_e.reshape(NT_PAD)

    nt = N_PAD // TILE
    ys = pl.pallas_call(
        _ffn_body,
        grid_spec=pltpu.PrefetchScalarGridSpec(
            num_scalar_prefetch=1,
            grid=(nt,),
            in_specs=[
                pl.BlockSpec((TILE, D), lambda i, te: (i, 0)),
                pl.BlockSpec((1, D, D_FF), lambda i, te: (te[i], 0, 0)),
                pl.BlockSpec((1, 1, D_FF), lambda i, te: (te[i], 0, 0)),
                pl.BlockSpec((1, D_FF, D), lambda i, te: (te[i], 0, 0)),
                pl.BlockSpec((1, 1, D), lambda i, te: (te[i], 0, 0)),
                pl.BlockSpec((TILE, 1), lambda i, te: (i, 0)),
            ],
            out_specs=pl.BlockSpec((TILE, D), lambda i, te: (i, 0)),
        ),
        out_shape=jax.ShapeDtypeStruct((N_PAD, D), jnp.float32),
        compiler_params=pltpu.CompilerParams(
            dimension_semantics=("arbitrary",),
        ),
    )(tile_e, xs, W1, b1.reshape(E, 1, D_FF), W2, b2.reshape(E, 1, D),
      wsort.reshape(N_PAD, 1))

    out = _mk_combine(T, D, N_PAD)(ys, pos)
    return out.reshape(B, S, D)


# speedup vs baseline: 1.0440x; 1.0440x over previous
import functools

import jax
import jax.numpy as jnp
from jax import lax
from jax.experimental import pallas as pl
from jax.experimental.pallas import tpu as pltpu
from jax.experimental.pallas import tpu_sc as plsc

TILE = 512


def _gate_rank_body(x_ref, Wg_ref, bg_ref, tri_ref,
                    e1_ref, e2_ref, w1_ref, w2_ref, r1_ref, r2_ref,
                    off_ref, te_ref, carry_ref, oh2_ref, *, n_exp, tile, nb):
    i = pl.program_id(0)

    @pl.when(i == 0)
    def _init():
        carry_ref[...] = jnp.zeros_like(carry_ref)

    blk = x_ref.shape[0]

    @pl.when(i < nb)
    def _pass1():
        x = x_ref[...]
        logits = jnp.dot(x, Wg_ref[...], preferred_element_type=jnp.float32)
        logits = logits + bg_ref[...]
        iota_e = lax.broadcasted_iota(jnp.int32, logits.shape, 1)
        a1 = jnp.argmax(logits, axis=-1)[:, None]
        m1 = jnp.max(logits, axis=-1, keepdims=True)
        l2 = jnp.where(iota_e == a1, -jnp.inf, logits)
        a2 = jnp.argmax(l2, axis=-1)[:, None]
        m2 = jnp.max(l2, axis=-1, keepdims=True)
        w1 = 1.0 / (1.0 + jnp.exp(m2 - m1))
        e1_ref[...] = a1
        e2_ref[...] = a2
        w1_ref[...] = w1
        w2_ref[...] = 1.0 - w1
        oh1 = (iota_e == a1).astype(jnp.float32)
        oh2_ref[pl.ds(i * blk, blk), :] = (iota_e == a2).astype(jnp.float32)
        cum = jnp.dot(tri_ref[...], oh1, preferred_element_type=jnp.float32)
        cum = cum + carry_ref[...]
        r1_ref[...] = (jnp.sum(cum * oh1, axis=1, keepdims=True) - 1.0
                       ).astype(jnp.int32)
        carry_ref[...] = cum[-1:, :]

    @pl.when(i >= nb)
    def _pass2():
        oh2 = oh2_ref[pl.ds((i - nb) * blk, blk), :]
        cum = jnp.dot(tri_ref[...], oh2, preferred_element_type=jnp.float32)
        cum = cum + carry_ref[...]
        r2_ref[...] = (jnp.sum(cum * oh2, axis=1, keepdims=True) - 1.0
                       ).astype(jnp.int32)
        carry_new = cum[-1:, :]
        carry_ref[...] = carry_new
        pc = jnp.floor((carry_new + (tile - 1)) / tile) * tile
        r8 = lax.broadcasted_iota(jnp.int32, (n_exp, n_exp), 0)
        c8 = lax.broadcasted_iota(jnp.int32, (n_exp, n_exp), 1)
        incl_m = (r8 <= c8).astype(jnp.float32)
        incl = jnp.dot(pc, incl_m, preferred_element_type=jnp.float32)
        off_ref[...] = jnp.broadcast_to(incl - pc, off_ref.shape).astype(jnp.int32)
        nt_pad = te_ref.shape[0]
        ts = lax.broadcasted_iota(jnp.int32, (nt_pad, n_exp), 0) * tile
        acc = jnp.sum((ts.astype(jnp.float32) >= incl).astype(jnp.float32),
                      axis=1, keepdims=True).astype(jnp.int32)
        te_ref[...] = jnp.minimum(acc, n_exp - 1)


def _mk_route_build(P, T, n_pad, n_tiles, n_exp):
    mesh = plsc.VectorSubcoreMesh(core_axis_name="c", subcore_axis_name="s")

    @functools.partial(
        pl.kernel,
        mesh=mesh,
        out_type=[
            jax.ShapeDtypeStruct((n_pad,), jnp.int32),
            jax.ShapeDtypeStruct((n_pad,), jnp.float32),
            jax.ShapeDtypeStruct((P,), jnp.int32),
        ],
        compiler_params=pltpu.CompilerParams(needs_layout_passes=False),
        scratch_types=[
            pltpu.VMEM((P,), jnp.int32),
            pltpu.VMEM((P,), jnp.int32),
            pltpu.VMEM((P,), jnp.float32),
            pltpu.VMEM((P,), jnp.int32),
            pltpu.VMEM((n_pad,), jnp.int32),
            pltpu.VMEM((n_pad,), jnp.float32),
            pltpu.VMEM((16,), jnp.int32),
        ],
    )
    def route_build(e_hbm, r_hbm, w_hbm, off_hbm,
                    si_hbm, ws_hbm, pos_hbm,
                    e_v, r_v, w_v, pos_v, si_v, ws_v, off_v):
        wid = lax.axis_index("s") * 2 + lax.axis_index("c")

        @pl.when(wid == 0)
        def _work():
            pltpu.sync_copy(e_hbm, e_v)
            pltpu.sync_copy(r_hbm, r_v)
            pltpu.sync_copy(w_hbm, w_v)
            pltpu.sync_copy(off_hbm, off_v)

            lanes = lax.iota(jnp.int32, 16)

            def _zero(i, _):
                si_v[pl.ds(i * 16, 16)] = (i * 16 + lanes) & (T - 1)
                ws_v[pl.ds(i * 16, 16)] = jnp.zeros((16,), jnp.float32)
                return 0
            lax.fori_loop(0, n_pad // 16, _zero, 0)

            def _scatter(i, _):
                base = i * 16
                ve = e_v[pl.ds(base, 16)]
                vr = r_v[pl.ds(base, 16)]
                vw = w_v[pl.ds(base, 16)]
                voff = plsc.load_gather(off_v, [ve])
                vpos = voff + vr
                pos_v[pl.ds(base, 16)] = vpos
                vp = base + lanes
                vt = vp - jnp.where(vp >= T, jnp.int32(T), 0)
                plsc.store_scatter(si_v, [vpos], vt)
                plsc.store_scatter(ws_v, [vpos], vw)
                return 0
            lax.fori_loop(0, P // 16, _scatter, 0)

            pltpu.sync_copy(si_v, si_hbm)
            pltpu.sync_copy(ws_v, ws_hbm)
            pltpu.sync_copy(pos_v, pos_hbm)

    return route_build


def _mk_gather(T, D, n_pad, dtype=jnp.float32):
    mesh = plsc.VectorSubcoreMesh(core_axis_name="c", subcore_axis_name="s")
    NW = 32
    b_per_w = n_pad // NW
    CH = 80
    n_ch = b_per_w // CH

    @functools.partial(
        pl.kernel,
        mesh=mesh,
        out_type=jax.ShapeDtypeStruct((n_pad, D), dtype),
        compiler_params=pltpu.CompilerParams(needs_layout_passes=False),
        scratch_types=[
            pltpu.VMEM((b_per_w,), jnp.int32),
            pltpu.VMEM((CH, D), dtype),
            pltpu.VMEM((CH, D), dtype),
            pltpu.SemaphoreType.DMA,
            pltpu.SemaphoreType.DMA,
        ],
    )
    def gather_rows(x_hbm, idx_hbm, out_hbm, idx_v, b0_v, b1_v, s0, s1):
        wid = lax.axis_index("s") * 2 + lax.axis_index("c")
        base = wid * b_per_w
        pltpu.sync_copy(idx_hbm.at[pl.ds(base, b_per_w)], idx_v)

        def _chunk(c, _):
            cp0 = pltpu.async_copy(
                x_hbm.at[idx_v.at[pl.ds(2 * c * CH, CH)]], b0_v, s0)
            cp1 = pltpu.async_copy(
                x_hbm.at[idx_v.at[pl.ds((2 * c + 1) * CH, CH)]], b1_v, s1)
            cp0.wait()
            cp1.wait()
            pltpu.sync_copy(b0_v, out_hbm.at[pl.ds(base + 2 * c * CH, CH)])
            pltpu.sync_copy(b1_v, out_hbm.at[pl.ds(base + (2 * c + 1) * CH, CH)])
            return 0
        lax.fori_loop(0, n_ch // 2, _chunk, 0)

    return gather_rows


def _ffn_body(te_ref, xs_ref, W1_ref, b1_ref, W2_ref, b2_ref, ws_ref, ys_ref):
    x = xs_ref[...]
    h = jnp.dot(x, W1_ref[0], preferred_element_type=jnp.float32) + b1_ref[0]
    h = h * (1.0 / (1.0 + jnp.exp(-h)))
    y = jnp.dot(h, W2_ref[0], preferred_element_type=jnp.float32) + b2_ref[0]
    ys_ref[...] = y * ws_ref[...]


def _mk_combine(T, D, n_pad):
    mesh = plsc.VectorSubcoreMesh(core_axis_name="c", subcore_axis_name="s")
    NW = 32
    t_per_w = T // NW
    CH = 64
    n_ch = t_per_w // CH

    @functools.partial(
        pl.kernel,
        mesh=mesh,
        out_type=jax.ShapeDtypeStruct((T, D), jnp.float32),
        compiler_params=pltpu.CompilerParams(needs_layout_passes=False),
        scratch_types=[
            pltpu.VMEM((t_per_w,), jnp.int32),
            pltpu.VMEM((t_per_w,), jnp.int32),
            pltpu.VMEM((CH, D), jnp.float32),
            pltpu.VMEM((CH, D), jnp.float32),
            pltpu.SemaphoreType.DMA,
            pltpu.SemaphoreType.DMA,
        ],
    )
    def combine(ys_hbm, pos_hbm, out_hbm, i0_v, i1_v, b0_v, b1_v, s0, s1):
        wid = lax.axis_index("s") * 2 + lax.axis_index("c")
        base = wid * t_per_w
        pltpu.sync_copy(pos_hbm.at[pl.ds(base, t_per_w)], i0_v)
        pltpu.sync_copy(pos_hbm.at[pl.ds(T + base, t_per_w)], i1_v)

        def _chunk(c, _):
            cp0 = pltpu.async_copy(ys_hbm.at[i0_v.at[pl.ds(c * CH, CH)]], b0_v, s0)
            cp1 = pltpu.async_copy(ys_hbm.at[i1_v.at[pl.ds(c * CH, CH)]], b1_v, s1)
            cp0.wait()
            cp1.wait()

            def _row(r, _):
                for col in range(D // 16):
                    sl = pl.ds(col * 16, 16)
                    b0_v[r, sl] = b0_v[r, sl] + b1_v[r, sl]
                return 0
            lax.fori_loop(0, CH, _row, 0)
            pltpu.sync_copy(b0_v, out_hbm.at[pl.ds(base + c * CH, CH)])
            return 0
        lax.fori_loop(0, n_ch, _chunk, 0)

    return combine


def kernel(inputs, Wg, bg, W1, b1, W2, b2):
    B, S, D = inputs.shape
    E = Wg.shape[1]
    D_FF = W1.shape[2]
    T = B * S
    K = 2
    P = K * T
    N_TILES_MAX = (P + E * TILE) // TILE
    N_PAD = N_TILES_MAX * TILE
    x = inputs.reshape(T, D)

    BLK = 512
    NB = T // BLK
    NT_PAD = ((N_TILES_MAX + 15) // 16) * 16
    tri = jnp.tril(jnp.ones((BLK, BLK), jnp.float32))
    blk_map = lambda i: (jnp.minimum(i, NB - 1), 0)
    blk2_map = lambda i: (jnp.maximum(i - NB, 0), 0)
    fix_map = lambda i: (0, 0)
    e1, e2, w1, w2, rank1, rank2, off, tile_e = pl.pallas_call(
        functools.partial(_gate_rank_body, n_exp=E, tile=TILE, nb=NB),
        grid=(2 * NB,),
        in_specs=[
            pl.BlockSpec((BLK, D), blk_map),
            pl.BlockSpec((D, E), fix_map),
            pl.BlockSpec((1, E), fix_map),
            pl.BlockSpec((BLK, BLK), fix_map),
        ],
        out_specs=[
            pl.BlockSpec((BLK, 1), blk_map),
            pl.BlockSpec((BLK, 1), blk_map),
            pl.BlockSpec((BLK, 1), blk_map),
            pl.BlockSpec((BLK, 1), blk_map),
            pl.BlockSpec((BLK, 1), blk_map),
            pl.BlockSpec((BLK, 1), blk2_map),
            pl.BlockSpec((8, E), fix_map),
            pl.BlockSpec((NT_PAD, 1), fix_map),
        ],
        out_shape=[
            jax.ShapeDtypeStruct((T, 1), jnp.int32),
            jax.ShapeDtypeStruct((T, 1), jnp.int32),
            jax.ShapeDtypeStruct((T, 1), jnp.float32),
            jax.ShapeDtypeStruct((T, 1), jnp.float32),
            jax.ShapeDtypeStruct((T, 1), jnp.int32),
            jax.ShapeDtypeStruct((T, 1), jnp.int32),
            jax.ShapeDtypeStruct((8, E), jnp.int32),
            jax.ShapeDtypeStruct((NT_PAD, 1), jnp.int32),
        ],
        scratch_shapes=[
            pltpu.VMEM((1, E), jnp.float32),
            pltpu.VMEM((T, E), jnp.float32),
        ],
        compiler_params=pltpu.CompilerParams(
            dimension_semantics=("arbitrary",),
        ),
    )(x, Wg, bg.reshape(1, E), tri)

    e_all = jnp.concatenate([e1, e2], axis=0)
    w_all = jnp.concatenate([w1, w2], axis=0)
    rank = jnp.concatenate([rank1, rank2], axis=0)

    off16 = jnp.pad(off[0], (0, 16 - E))

    route = _mk_route_build(P, T, N_PAD, N_TILES_MAX, E)
    src_idx, wsort, pos = route(
        e_all.reshape(P), rank.reshape(P), w_all.reshape(P), off16)
    tile_e = tile_e.reshape(NT_PAD)

    xs = _mk_gather(T, D, N_PAD)(x, src_idx)

    nt = N_PAD // TILE
    ys = pl.pallas_call(
        _ffn_body,
        grid_spec=pltpu.PrefetchScalarGridSpec(
            num_scalar_prefetch=1,
            grid=(nt,),
            in_specs=[
                pl.BlockSpec((TILE, D), lambda i, te: (i, 0)),
                pl.BlockSpec((1, D, D_FF), lambda i, te: (te[i], 0, 0)),
                pl.BlockSpec((1, 1, D_FF), lambda i, te: (te[i], 0, 0)),
                pl.BlockSpec((1, D_FF, D), lambda i, te: (te[i], 0, 0)),
                pl.BlockSpec((1, 1, D), lambda i, te: (te[i], 0, 0)),
                pl.BlockSpec((TILE, 1), lambda i, te: (i, 0)),
            ],
            out_specs=pl.BlockSpec((TILE, D), lambda i, te: (i, 0)),
        ),
        out_shape=jax.ShapeDtypeStruct((N_PAD, D), jnp.float32),
        compiler_params=pltpu.CompilerParams(
            dimension_semantics=("arbitrary",),
        ),
    )(tile_e, xs, W1, b1.reshape(E, 1, D_FF), W2, b2.reshape(E, 1, D),
      wsort.reshape(N_PAD, 1))

    out = _mk_combine(T, D, N_PAD)(ys, pos)
    return out.reshape(B, S, D)
